# baseline (speedup 1.0000x reference)
import functools

import jax
import jax.numpy as jnp
from jax import lax
from jax.experimental import pallas as pl
from jax.experimental.pallas import tpu as pltpu
from jax.experimental.pallas import tpu_sc as plsc

N = 2048
ROWS = 256
NC, NS, L = 2, 16, 16
NW = NC * NS
RPW = ROWS // NW
EPS = 1e-8
DT = 1e-3
GROUPS = N // L
BS_STEPS = 12


def _prep_body(x_ref, y_ref, xt_ref, yt_ref):
    m = jnp.minimum(jnp.min(x_ref[...]), jnp.min(y_ref[...]))
    xt_ref[...] = jnp.transpose(x_ref[...], (0, 2, 1)) - m + EPS
    yt_ref[...] = jnp.transpose(y_ref[...], (0, 2, 1)) - m + EPS


def _cumsum_inplace(ref):
    def chunk2(i, carry):
        c0 = ref[pl.ds((2 * i) * L, L)]
        c1 = ref[pl.ds((2 * i + 1) * L, L)]
        s0 = plsc.cumsum(c0)
        s1 = plsc.cumsum(c1)
        cs0 = s0 + carry
        carry1 = jnp.max(cs0)
        cs1 = s1 + carry1
        ref[pl.ds((2 * i) * L, L)] = cs0
        ref[pl.ds((2 * i + 1) * L, L)] = cs1
        return jnp.max(cs1)
    return lax.fori_loop(0, GROUPS // 2, chunk2, jnp.float32(0.0))


_SHIFT_IDX = None


_LANES = None


def _both_directions(a_ref, b_ref, r_ab, r_ba):
    lanes = lax.iota(jnp.int32, L)
    shift_idx = jnp.maximum(lanes - 1, 0)
    dnums = lax.GatherDimensionNumbers(
        offset_dims=(), collapsed_slice_dims=(0,), start_index_map=(0,)
    )

    def _lane_shift(q):
        return lax.gather(
            q, shift_idx[:, None], dnums, slice_sizes=(1,),
            mode=lax.GatherScatterMode.PROMISE_IN_BOUNDS,
        )

    def make_chain(q_ref, t_ref, r_qt, right, g):
        q = q_ref[pl.ds(g * L, L)]
        qs = q * r_qt
        lo = jnp.zeros((L,), jnp.int32)
        hi = jnp.full((L,), N, jnp.int32)

        def step(lo, hi):
            mid = lax.shift_right_logical(lo + hi, 1)
            tv = plsc.load_gather(t_ref, [jnp.minimum(mid, N - 1)])
            pred = (tv <= qs) if right else (tv < qs)
            return jnp.where(pred, mid + 1, lo), jnp.where(pred, hi, mid)

        def finish(lo, r_tq, carry_q, g):
            c = lo
            tprev = plsc.load_gather(t_ref, [jnp.maximum(c - 1, 0)])
            tprev = jnp.where(c > 0, tprev * r_tq, 0.0)
            qprev = _lane_shift(q)
            qprev = jnp.where(lanes == 0, carry_q, qprev)
            ivec = g * L + lanes
            delta = q - jnp.maximum(qprev, tprev)
            di = (ivec - jnp.minimum(c, N - 1)).astype(jnp.float32)
            return delta * di * di, jnp.max(q)

        return (lo, hi), step, finish

    def body(i, state):
        acc_a, acc_b, cqa, cqb = state
        g0 = 2 * i
        g1 = 2 * i + 1
        sa0, step_a0, fin_a0 = make_chain(a_ref, b_ref, r_ab, False, g0)
        sa1, step_a1, fin_a1 = make_chain(a_ref, b_ref, r_ab, False, g1)

        def steps_a(_, st):
            (a0l, a0h), (a1l, a1h) = st
            return step_a0(a0l, a0h), step_a1(a1l, a1h)

        (a0l, _), (a1l, _) = lax.fori_loop(0, BS_STEPS, steps_a, (sa0, sa1))
        wa0, cqa = fin_a0(a0l, r_ba, cqa, g0)
        wa1, cqa = fin_a1(a1l, r_ba, cqa, g1)

        sb0, step_b0, fin_b0 = make_chain(b_ref, a_ref, r_ba, True, g0)
        sb1, step_b1, fin_b1 = make_chain(b_ref, a_ref, r_ba, True, g1)

        def steps_b(_, st):
            (b0l, b0h), (b1l, b1h) = st
            return step_b0(b0l, b0h), step_b1(b1l, b1h)

        (b0l, _), (b1l, _) = lax.fori_loop(0, BS_STEPS, steps_b, (sb0, sb1))
        wb0, cqb = fin_b0(b0l, r_ab, cqb, g0)
        wb1, cqb = fin_b1(b1l, r_ab, cqb, g1)
        return acc_a + wa0 + wa1, acc_b + wb0 + wb1, cqa, cqb

    z = jnp.zeros((L,), jnp.float32)
    acc_a, acc_b, _, _ = lax.fori_loop(
        0, GROUPS // 2, body, (z, z, jnp.float32(0.0), jnp.float32(0.0))
    )
    return acc_a, acc_b


def _sc_body(xt_hbm, yt_hbm, out_hbm, a_v, b_v, w_v):
    wid = lax.axis_index("s") * NC + lax.axis_index("c")
    lanes = lax.iota(jnp.int32, L)
    ones = jnp.ones((L,), jnp.float32)

    def row_body(r, wvec):
        row = wid * RPW + r
        pltpu.sync_copy(xt_hbm.at[pl.ds(row * N, N)], a_v)
        pltpu.sync_copy(yt_hbm.at[pl.ds(row * N, N)], b_v)
        ta = _cumsum_inplace(a_v)
        tb = _cumsum_inplace(b_v)
        ta_v = jnp.full((L,), ta, jnp.float32)
        tb_v = jnp.full((L,), tb, jnp.float32)
        r_ab = tb_v / ta_v
        r_ba = ta_v / tb_v
        acc_a, acc_b = _both_directions(a_v, b_v, r_ab, r_ba)
        w = jnp.sum(acc_a / ta_v + acc_b / tb_v) * (DT * DT)
        return jnp.where(lanes == r, w, wvec)

    wvec = lax.fori_loop(0, RPW, row_body, jnp.zeros((L,), jnp.float32))
    w_v[...] = wvec
    pltpu.sync_copy(w_v, out_hbm.at[wid])


def _sum_body(p_ref, o_ref):
    o_ref[...] = jnp.full((1, 1), jnp.sum(p_ref[...]), jnp.float32)


@jax.jit
def kernel(x, y):
    nb, nt, nr, nc = x.shape
    x2 = x.reshape(nb, nt, nr * nc)
    y2 = y.reshape(nb, nt, nr * nc)

    xt, yt = pl.pallas_call(
        _prep_body,
        out_shape=[
            jax.ShapeDtypeStruct((nb, nr * nc, nt), jnp.float32),
            jax.ShapeDtypeStruct((nb, nr * nc, nt), jnp.float32),
        ],
    )(x2, y2)

    mesh = plsc.VectorSubcoreMesh(
        core_axis_name="c", subcore_axis_name="s", num_cores=NC, num_subcores=NS
    )
    partials = pl.kernel(
        _sc_body,
        out_type=jax.ShapeDtypeStruct((NW, L), jnp.float32),
        mesh=mesh,
        compiler_params=pltpu.CompilerParams(needs_layout_passes=False),
        scratch_types=[
            pltpu.VMEM((N,), jnp.float32),
            pltpu.VMEM((N,), jnp.float32),
            pltpu.VMEM((L,), jnp.float32),
        ],
    )(xt.reshape(ROWS * N), yt.reshape(ROWS * N))

    loss = pl.pallas_call(
        _sum_body,
        out_shape=jax.ShapeDtypeStruct((1, 1), jnp.float32),
    )(partials.reshape(4, 128))
    return loss[0, 0]


# speedup vs baseline: 1.6128x; 1.6128x over previous
import functools

import jax
import jax.numpy as jnp
from jax import lax
from jax.experimental import pallas as pl
from jax.experimental.pallas import tpu as pltpu
from jax.experimental.pallas import tpu_sc as plsc

N = 2048
ROWS = 256
NC, NS, L = 2, 16, 16
NW = NC * NS
RPW = ROWS // NW
EPS = 1e-8
DT = 1e-3
GROUPS = N // L
BS_STEPS = 12


def _prep_body(x_ref, y_ref, xt_ref, yt_ref):
    m = jnp.minimum(jnp.min(x_ref[...]), jnp.min(y_ref[...]))
    xt_ref[...] = jnp.transpose(x_ref[...], (0, 2, 1)) - m + EPS
    yt_ref[...] = jnp.transpose(y_ref[...], (0, 2, 1)) - m + EPS


def _cumsum_inplace(ref):
    def chunk2(i, carry):
        c0 = ref[pl.ds((2 * i) * L, L)]
        c1 = ref[pl.ds((2 * i + 1) * L, L)]
        s0 = plsc.cumsum(c0)
        s1 = plsc.cumsum(c1)
        cs0 = s0 + carry
        carry1 = jnp.max(cs0)
        cs1 = s1 + carry1
        ref[pl.ds((2 * i) * L, L)] = cs0
        ref[pl.ds((2 * i + 1) * L, L)] = cs1
        return jnp.max(cs1)
    return lax.fori_loop(0, GROUPS // 2, chunk2, jnp.float32(0.0))


_SHIFT_IDX = None


def _direction(q_ref, t_ref, r_qt, r_tq, right):
    lanes = lax.iota(jnp.int32, L)
    shift_idx = jnp.maximum(lanes - 1, 0)
    dnums = lax.GatherDimensionNumbers(
        offset_dims=(), collapsed_slice_dims=(0,), start_index_map=(0,)
    )

    def _lane_shift(q):
        return lax.gather(
            q, shift_idx[:, None], dnums, slice_sizes=(1,),
            mode=lax.GatherScatterMode.PROMISE_IN_BOUNDS,
        )

    def one_group(g, carry_q):
        q = q_ref[pl.ds(g * L, L)]
        qs = q * r_qt
        lo = jnp.zeros((L,), jnp.int32)
        hi = jnp.full((L,), N, jnp.int32)

        def step(_, lohi):
            lo, hi = lohi
            mid = lax.shift_right_logical(lo + hi, 1)
            tv = plsc.load_gather(t_ref, [jnp.minimum(mid, N - 1)])
            pred = (tv <= qs) if right else (tv < qs)
            return jnp.where(pred, mid + 1, lo), jnp.where(pred, hi, mid)

        c, _ = lax.fori_loop(0, BS_STEPS, step, (lo, hi))
        tprev = plsc.load_gather(t_ref, [jnp.maximum(c - 1, 0)])
        tprev = jnp.where(c > 0, tprev * r_tq, 0.0)
        qprev = _lane_shift(q)
        qprev = jnp.where(lanes == 0, carry_q, qprev)
        ivec = g * L + lanes
        delta = q - jnp.maximum(qprev, tprev)
        di = (ivec - jnp.minimum(c, N - 1)).astype(jnp.float32)
        return delta * di * di, jnp.max(q)

    def group2(i, state):
        acc0, acc1, carry_q = state
        w0, carry_q = one_group(2 * i, carry_q)
        w1, carry_q = one_group(2 * i + 1, carry_q)
        return acc0 + w0, acc1 + w1, carry_q

    z = jnp.zeros((L,), jnp.float32)
    acc0, acc1, _ = lax.fori_loop(
        0, GROUPS // 2, group2, (z, z, jnp.float32(0.0))
    )
    return acc0 + acc1


def _sc_body(xt_hbm, yt_hbm, out_hbm, a_v, b_v, w_v):
    wid = lax.axis_index("s") * NC + lax.axis_index("c")
    lanes = lax.iota(jnp.int32, L)
    ones = jnp.ones((L,), jnp.float32)

    def row_body(r, wvec):
        row = wid * RPW + r
        pltpu.sync_copy(xt_hbm.at[pl.ds(row * N, N)], a_v)
        pltpu.sync_copy(yt_hbm.at[pl.ds(row * N, N)], b_v)
        ta = _cumsum_inplace(a_v)
        tb = _cumsum_inplace(b_v)
        ta_v = jnp.full((L,), ta, jnp.float32)
        tb_v = jnp.full((L,), tb, jnp.float32)
        r_ab = tb_v / ta_v
        r_ba = ta_v / tb_v
        acc = (_direction(a_v, b_v, r_ab, r_ba, right=False) / ta_v
               + _direction(b_v, a_v, r_ba, r_ab, right=True) / tb_v)
        w = jnp.sum(acc) * (DT * DT)
        return jnp.where(lanes == r, w, wvec)

    wvec = lax.fori_loop(0, RPW, row_body, jnp.zeros((L,), jnp.float32))
    w_v[...] = wvec
    pltpu.sync_copy(w_v, out_hbm.at[wid])


def _sum_body(p_ref, o_ref):
    o_ref[...] = jnp.full((1, 1), jnp.sum(p_ref[...]), jnp.float32)


@jax.jit
def kernel(x, y):
    nb, nt, nr, nc = x.shape
    x2 = x.reshape(nb, nt, nr * nc)
    y2 = y.reshape(nb, nt, nr * nc)

    xt, yt = pl.pallas_call(
        _prep_body,
        out_shape=[
            jax.ShapeDtypeStruct((nb, nr * nc, nt), jnp.float32),
            jax.ShapeDtypeStruct((nb, nr * nc, nt), jnp.float32),
        ],
    )(x2, y2)

    mesh = plsc.VectorSubcoreMesh(
        core_axis_name="c", subcore_axis_name="s", num_cores=NC, num_subcores=NS
    )
    partials = pl.kernel(
        _sc_body,
        out_type=jax.ShapeDtypeStruct((NW, L), jnp.float32),
        mesh=mesh,
        compiler_params=pltpu.CompilerParams(needs_layout_passes=False),
        scratch_types=[
            pltpu.VMEM((N,), jnp.float32),
            pltpu.VMEM((N,), jnp.float32),
            pltpu.VMEM((L,), jnp.float32),
        ],
    )(xt.reshape(ROWS * N), yt.reshape(ROWS * N))

    loss = pl.pallas_call(
        _sum_body,
        out_shape=jax.ShapeDtypeStruct((1, 1), jnp.float32),
    )(partials.reshape(4, 128))
    return loss[0, 0]


# speedup vs baseline: 1.8104x; 1.1225x over previous
import functools

import jax
import jax.numpy as jnp
from jax import lax
from jax.experimental import pallas as pl
from jax.experimental.pallas import tpu as pltpu
from jax.experimental.pallas import tpu_sc as plsc

N = 2048
ROWS = 256
NC, NS, L = 2, 16, 16
NW = NC * NS
RPW = ROWS // NW
EPS = 1e-8
DT = 1e-3
GROUPS = N // L
BS_STEPS = 12


def _prep_body(x_ref, y_ref, xt_ref, yt_ref):
    m = jnp.minimum(jnp.min(x_ref[...]), jnp.min(y_ref[...]))
    xt_ref[...] = jnp.transpose(x_ref[...], (0, 2, 1)) - m + EPS
    yt_ref[...] = jnp.transpose(y_ref[...], (0, 2, 1)) - m + EPS


_DNUMS = lax.GatherDimensionNumbers(
    offset_dims=(), collapsed_slice_dims=(0,), start_index_map=(0,)
)


def _vgather(v, idx):
    return lax.gather(
        v, idx[:, None], _DNUMS, slice_sizes=(1,),
        mode=lax.GatherScatterMode.PROMISE_IN_BOUNDS,
    )


def _bcast_last(v):
    return _vgather(v, jnp.full((L,), L - 1, jnp.int32))


def _cumsum_inplace(ref):
    def chunk2(i, carry):
        c0 = ref[pl.ds((2 * i) * L, L)]
        c1 = ref[pl.ds((2 * i + 1) * L, L)]
        s0 = plsc.cumsum(c0)
        s1 = plsc.cumsum(c1)
        cs0 = s0 + carry
        cs1 = s1 + _bcast_last(cs0)
        ref[pl.ds((2 * i) * L, L)] = cs0
        ref[pl.ds((2 * i + 1) * L, L)] = cs1
        return _bcast_last(cs1)
    return lax.fori_loop(0, GROUPS // 2, chunk2, jnp.zeros((L,), jnp.float32))


_SHIFT_IDX = None


def _direction(q_ref, t_ref, r_qt, r_tq, right):
    lanes = lax.iota(jnp.int32, L)
    shift_idx = jnp.maximum(lanes - 1, 0)

    def _lane_shift(q):
        return _vgather(q, shift_idx)

    def one_group(g, carry_q):
        q = q_ref[pl.ds(g * L, L)]
        qs = q * r_qt
        lo = jnp.zeros((L,), jnp.int32)
        hi = jnp.full((L,), N, jnp.int32)

        def step(_, lohi):
            lo, hi = lohi
            mid = lax.shift_right_logical(lo + hi, 1)
            tv = plsc.load_gather(t_ref, [jnp.minimum(mid, N - 1)])
            pred = (tv <= qs) if right else (tv < qs)
            return jnp.where(pred, mid + 1, lo), jnp.where(pred, hi, mid)

        c, _ = lax.fori_loop(0, BS_STEPS, step, (lo, hi))
        tprev = plsc.load_gather(t_ref, [jnp.maximum(c - 1, 0)])
        tprev = jnp.where(c > 0, tprev * r_tq, 0.0)
        qprev = _lane_shift(q)
        qprev = jnp.where(lanes == 0, carry_q, qprev)
        ivec = g * L + lanes
        delta = q - jnp.maximum(qprev, tprev)
        di = (ivec - jnp.minimum(c, N - 1)).astype(jnp.float32)
        return delta * di * di, _bcast_last(q)

    def group2(i, state):
        acc0, acc1, carry_q = state
        w0, carry_q = one_group(2 * i, carry_q)
        w1, carry_q = one_group(2 * i + 1, carry_q)
        return acc0 + w0, acc1 + w1, carry_q

    z = jnp.zeros((L,), jnp.float32)
    acc0, acc1, _ = lax.fori_loop(0, GROUPS // 2, group2, (z, z, z))
    return acc0 + acc1


def _sc_body(xt_hbm, yt_hbm, out_hbm, a_v, b_v, w_v):
    wid = lax.axis_index("s") * NC + lax.axis_index("c")
    lanes = lax.iota(jnp.int32, L)
    ones = jnp.ones((L,), jnp.float32)

    def row_body(r, wvec):
        row = wid * RPW + r
        pltpu.sync_copy(xt_hbm.at[pl.ds(row * N, N)], a_v)
        pltpu.sync_copy(yt_hbm.at[pl.ds(row * N, N)], b_v)
        ta_v = _cumsum_inplace(a_v)
        tb_v = _cumsum_inplace(b_v)
        r_ab = tb_v / ta_v
        r_ba = ta_v / tb_v
        acc = (_direction(a_v, b_v, r_ab, r_ba, right=False) / ta_v
               + _direction(b_v, a_v, r_ba, r_ab, right=True) / tb_v)
        w = jnp.sum(acc) * (DT * DT)
        return jnp.where(lanes == r, w, wvec)

    wvec = lax.fori_loop(0, RPW, row_body, jnp.zeros((L,), jnp.float32))
    w_v[...] = wvec
    pltpu.sync_copy(w_v, out_hbm.at[wid])


def _sum_body(p_ref, o_ref):
    o_ref[...] = jnp.full((1, 1), jnp.sum(p_ref[...]), jnp.float32)


@jax.jit
def kernel(x, y):
    nb, nt, nr, nc = x.shape
    x2 = x.reshape(nb, nt, nr * nc)
    y2 = y.reshape(nb, nt, nr * nc)

    xt, yt = pl.pallas_call(
        _prep_body,
        out_shape=[
            jax.ShapeDtypeStruct((nb, nr * nc, nt), jnp.float32),
            jax.ShapeDtypeStruct((nb, nr * nc, nt), jnp.float32),
        ],
    )(x2, y2)

    mesh = plsc.VectorSubcoreMesh(
        core_axis_name="c", subcore_axis_name="s", num_cores=NC, num_subcores=NS
    )
    partials = pl.kernel(
        _sc_body,
        out_type=jax.ShapeDtypeStruct((NW, L), jnp.float32),
        mesh=mesh,
        compiler_params=pltpu.CompilerParams(needs_layout_passes=False),
        scratch_types=[
            pltpu.VMEM((N,), jnp.float32),
            pltpu.VMEM((N,), jnp.float32),
            pltpu.VMEM((L,), jnp.float32),
        ],
    )(xt.reshape(ROWS * N), yt.reshape(ROWS * N))

    loss = pl.pallas_call(
        _sum_body,
        out_shape=jax.ShapeDtypeStruct((1, 1), jnp.float32),
    )(partials.reshape(4, 128))
    return loss[0, 0]
